# baseline (speedup 1.0000x reference)
import jax
import jax.numpy as jnp
from jax.experimental import pallas as pl


def kernel(input, table):
    B, S, D = input.shape
    BB = 32

    def body(table_ref, out_ref):
        emb = table_ref[:S, :]
        out_ref[...] = jnp.broadcast_to(emb[None], (BB, S, D))

    out = pl.pallas_call(
        body,
        grid=(B // BB,),
        in_specs=[pl.BlockSpec((table.shape[0], D), lambda i: (0, 0))],
        out_specs=pl.BlockSpec((BB, S, D), lambda i: (i, 0, 0)),
        out_shape=jax.ShapeDtypeStruct((B, S, D), jnp.float32),
    )(table)
    return out
